# baseline (device time: 129542 ns/iter reference)
import jax
import jax.numpy as jnp
from jax import lax
from jax.experimental import pallas as pl
from jax.experimental.pallas import tpu as pltpu

N_DEV = 16
HR = 8
HL = 7
B_LOC = 2
SQ = 256
SKV = 256
HQ_LOC = 4
DH = 64
D_MODEL = 512
HD_LOC = HQ_LOC * DH


def kernel(x, Wq, K_ext, V_ext, Wo):
    my = lax.axis_index("i")
    WoT = Wo.T
    x2 = x.reshape(B_LOC * SQ, D_MODEL)
    K_loc = lax.dynamic_slice_in_dim(K_ext, my * B_LOC, B_LOC, 0).transpose(0, 2, 1, 3)
    V_loc = lax.dynamic_slice_in_dim(V_ext, my * B_LOC, B_LOC, 0).transpose(0, 2, 1, 3)

    def body(x_ref, wq_ref, k_hbm, v_hbm, wot_ref, out_ref,
             gR, gL, ctx_s, k_buf, v_buf,
             sR, rR, sL, rL, kv_sem):
        my_pos = lax.axis_index("i")
        left = lax.rem(my_pos - 1 + N_DEV, N_DEV)
        right = lax.rem(my_pos + 1, N_DEV)

        barrier = pltpu.get_barrier_semaphore()
        pl.semaphore_signal(barrier, inc=1, device_id=(left,),
                            device_id_type=pl.DeviceIdType.MESH)
        pl.semaphore_signal(barrier, inc=1, device_id=(right,),
                            device_id_type=pl.DeviceIdType.MESH)
        pl.semaphore_wait(barrier, 2)

        qi = lax.broadcasted_iota(jnp.int32, (SQ, SKV), 0) // 64
        kj = lax.broadcasted_iota(jnp.int32, (SQ, SKV), 1) // 64
        mask = (qi == kj) | (kj == 0) | (lax.rem(qi + kj, 3) == 0)
        bias = jnp.where(mask, 0.0, -30.0).astype(jnp.float32)

        gR[0, 0, :, :] = wq_ref[:, :]
        gR[0, 1, :, :] = wot_ref[:, :]
        gL[0, 0, :, :] = wq_ref[:, :]
        gL[0, 1, :, :] = wot_ref[:, :]

        N_SEG = 4

        def _seg(buf, h, s):
            return buf.at[h, s // 2, pl.ds((s % 2) * 256, 256)]

        def rdR(h, s):
            return pltpu.make_async_remote_copy(
                src_ref=_seg(gR, h - 1, s), dst_ref=_seg(gR, h, s),
                send_sem=sR.at[h, s], recv_sem=rR.at[h, s],
                device_id=(right,), device_id_type=pl.DeviceIdType.MESH)

        def rdL(h, s):
            return pltpu.make_async_remote_copy(
                src_ref=_seg(gL, h - 1, s), dst_ref=_seg(gL, h, s),
                send_sem=sL.at[h, s], recv_sem=rL.at[h, s],
                device_id=(left,), device_id_type=pl.DeviceIdType.MESH)

        def kv_descr(d, h):
            src = lax.rem(my_pos + (h if d else -h) + N_DEV, N_DEV)
            g0 = src * HQ_LOC
            kidx = d * 2 + lax.rem(h, 2)
            ck = pltpu.make_async_copy(
                k_hbm.at[:, pl.ds(g0, HQ_LOC)], k_buf.at[kidx], kv_sem.at[kidx, 0])
            cv = pltpu.make_async_copy(
                v_hbm.at[:, pl.ds(g0, HQ_LOC)], v_buf.at[kidx], kv_sem.at[kidx, 1])
            return ck, cv

        def start_fetch(d, h):
            ck, cv = kv_descr(d, h)
            ck.start()
            cv.start()

        def wait_fetch(d, h):
            ck, cv = kv_descr(d, h)
            ck.wait()
            cv.wait()

        def compute(g_ref, h, kidx, first):
            wq_h = g_ref[h, 0]
            wot_h = g_ref[h, 1]
            k_blk = k_buf[kidx]
            v_blk = v_buf[kidx]
            q_2b = jnp.dot(x_ref[:, :], wq_h,
                           preferred_element_type=jnp.float32)
            for b in range(B_LOC):
                q_all = q_2b[b * SQ:(b + 1) * SQ, :]
                for g in range(HQ_LOC):
                    q = q_all[:, g * DH:(g + 1) * DH]
                    k = k_blk[b, g]
                    v = v_blk[b, g]
                    s = lax.dot_general(
                        q, k, (((1,), (1,)), ((), ())),
                        preferred_element_type=jnp.float32)
                    e = jnp.exp(s * 0.125 + bias)
                    recip = 1.0 / jnp.sum(e, axis=1, keepdims=True)
                    ctx_s[:, g * DH:(g + 1) * DH] = jnp.dot(
                        e, v, preferred_element_type=jnp.float32) * recip
                contrib = lax.dot_general(
                    ctx_s[:, :], wot_h, (((1,), (1,)), ((), ())),
                    preferred_element_type=jnp.float32)
                if first:
                    out_ref[b, :, :] = contrib
                else:
                    out_ref[b, :, :] = out_ref[b, :, :] + contrib

        for s in range(N_SEG):
            rdR(1, s).start()
            rdL(1, s).start()
        start_fetch(0, 0)
        wait_fetch(0, 0)
        start_fetch(0, 1)
        start_fetch(1, 1)
        compute(gR, 0, 0, first=True)

        def hop(h, carry):
            for s in range(N_SEG):
                rdR(h, s).wait_recv()
                rdR(h + 1, s).start()
                rdL(h, s).wait_recv()

                @pl.when(h < HL)
                def _():
                    rdL(h + 1, s).start()

            wait_fetch(0, h)
            wait_fetch(1, h)
            start_fetch(0, h + 1)

            @pl.when(h < HL)
            def _():
                start_fetch(1, h + 1)

            compute(gR, h, lax.rem(h, 2), first=False)
            compute(gL, h, 2 + lax.rem(h, 2), first=False)
            for s in range(N_SEG):
                rdR(h, s).wait_send()
                rdL(h, s).wait_send()
            return carry

        lax.fori_loop(1, HR, hop, 0)

        for s in range(N_SEG):
            rdR(HR, s).wait_recv()
        wait_fetch(0, HR)
        compute(gR, HR, lax.rem(HR, 2), first=False)
        for s in range(N_SEG):
            rdR(HR, s).wait_send()

    return pl.pallas_call(
        body,
        out_shape=jax.ShapeDtypeStruct((B_LOC, SQ, D_MODEL), jnp.float32),
        in_specs=[
            pl.BlockSpec(memory_space=pltpu.VMEM),
            pl.BlockSpec(memory_space=pltpu.VMEM),
            pl.BlockSpec(memory_space=pl.ANY),
            pl.BlockSpec(memory_space=pl.ANY),
            pl.BlockSpec(memory_space=pltpu.VMEM),
        ],
        out_specs=pl.BlockSpec(memory_space=pltpu.VMEM),
        scratch_shapes=[
            pltpu.VMEM((HR + 1, 2, D_MODEL, HD_LOC), jnp.float32),
            pltpu.VMEM((HL + 1, 2, D_MODEL, HD_LOC), jnp.float32),
            pltpu.VMEM((SQ, HD_LOC), jnp.float32),
            pltpu.VMEM((4, B_LOC, HQ_LOC, SKV, DH), jnp.float32),
            pltpu.VMEM((4, B_LOC, HQ_LOC, SKV, DH), jnp.float32),
            pltpu.SemaphoreType.DMA((HR + 1, 4)),
            pltpu.SemaphoreType.DMA((HR + 1, 4)),
            pltpu.SemaphoreType.DMA((HL + 1, 4)),
            pltpu.SemaphoreType.DMA((HL + 1, 4)),
            pltpu.SemaphoreType.DMA((4, 2)),
        ],
        compiler_params=pltpu.CompilerParams(collective_id=0),
    )(x2, Wq, K_loc, V_loc, WoT)
